# baseline (speedup 1.0000x reference)
import functools

import jax
import jax.numpy as jnp
from jax import lax
from jax.experimental import pallas as pl
from jax.experimental.pallas import tpu as pltpu
from jax.experimental.pallas import tpu_sc as plsc

N = 10000
E = 320000
D = 128
NC, NS, L = 2, 16, 16
NW = NC * NS
EPT = 10240
E_PAD = NW * EPT
CH = 64
SUP = 8
NSUP = EPT // (SUP * CH)
N_PAD = 10240
RPT = N_PAD // NS
HW = D // 2


def _proj_body(x_ref, wt_ref, a2_ref, h_ref, sds_ref):
    h = jnp.dot(x_ref[...], wt_ref[...], preferred_element_type=jnp.float32)
    h_ref[...] = h
    sds_ref[...] = jnp.dot(h, a2_ref[...], preferred_element_type=jnp.float32)


def _ed_body(ea_ref, bd_ref, ed_ref):
    ed_ref[...] = jnp.dot(ea_ref[...], bd_ref[...],
                          preferred_element_type=jnp.float32)


def _sc_body(h_hbm, sd_hbm, ss_hbm, dst_hbm, src_hbm, ed_hbm,
             zatt_hbm, zden_hbm,
             att_out, den_out,
             sd_v, ss_v, dst_v, src_v, ed_v, ex_v, rows_v, scaled_v,
             att_sh, den_sh,
             gsem0, gsem1, ssem0, ssem1, stsem):
    c = lax.axis_index("c")
    s = lax.axis_index("s")
    wid = s * NC + c

    pltpu.sync_copy(sd_hbm, sd_v)
    pltpu.sync_copy(ss_hbm, ss_v)
    pltpu.sync_copy(zatt_hbm.at[pl.ds(s * RPT, RPT)],
                    att_sh.at[pl.ds(s * RPT, RPT)])
    pltpu.sync_copy(zden_hbm.at[pl.ds(s * RPT, RPT)],
                    den_sh.at[pl.ds(s * RPT, RPT)])
    plsc.subcore_barrier()

    lane = lax.iota(jnp.int32, L)

    def stage_start(k, b):
        pltpu.async_copy(dst_hbm.at[wid, k], dst_v.at[b], stsem)
        pltpu.async_copy(src_hbm.at[wid, k], src_v.at[b], stsem)
        pltpu.async_copy(ed_hbm.at[wid, k], ed_v.at[b], stsem)

    def stage_wait(k, b):
        pltpu.make_async_copy(dst_hbm.at[wid, k], dst_v.at[b], stsem).wait()
        pltpu.make_async_copy(src_hbm.at[wid, k], src_v.at[b], stsem).wait()
        pltpu.make_async_copy(ed_hbm.at[wid, k], ed_v.at[b], stsem).wait()

    def gsem(p):
        return gsem0 if p == 0 else gsem1

    def ssem(p):
        return ssem0 if p == 0 else ssem1

    def compute_ex(k, b, j, p):
        for g in range(CH // L):
            di = dst_v[b, j, pl.ds(g * L, L)]
            si = src_v[b, j, pl.ds(g * L, L)]
            sc = (plsc.load_gather(sd_v, [di])
                  + plsc.load_gather(ss_v, [si])
                  + ed_v[b, j, pl.ds(g * L, L)])
            sc = jnp.where(sc >= 0.0, sc, 0.2 * sc)
            ex = jnp.exp(sc)
            gid = wid * EPT + (k * SUP + j) * CH + g * L + lane
            ex_v[p, pl.ds(g * L, L)] = jnp.where(gid < E, ex, 0.0)

    def gather_start(b, j, p):
        pltpu.async_copy(h_hbm.at[src_v.at[b, j]], rows_v.at[p], gsem(p))

    def gather_wait(b, j, p):
        pltpu.make_async_copy(
            h_hbm.at[src_v.at[b, j]], rows_v.at[p], gsem(p)).wait()

    def scatter_start(b, j, p):
        pltpu.async_copy(scaled_v.at[p], att_sh.at[dst_v.at[b, j]], ssem(p),
                         add=True)
        pltpu.async_copy(ex_v.at[p], den_sh.at[dst_v.at[b, j]], ssem(p),
                         add=True)

    def scatter_wait(b, j, p):
        pltpu.make_async_copy(
            scaled_v.at[p], att_sh.at[dst_v.at[b, j]], ssem(p)).wait()
        pltpu.make_async_copy(
            ex_v.at[p], den_sh.at[dst_v.at[b, j]], ssem(p)).wait()

    def scale(p):
        def edge_body(e, carry2):
            bv = plsc.load_gather(ex_v.at[p], [jnp.zeros((L,), jnp.int32) + e])
            for g in range(HW // L):
                v = rows_v[p, e, pl.ds(g * L, L)]
                lo = lax.bitcast_convert_type(v << 16, jnp.float32)
                hi = lax.bitcast_convert_type(v & jnp.int32(-65536),
                                              jnp.float32)
                scaled_v[p, e, pl.ds(g * L, L)] = lo * bv
                scaled_v[p, e, pl.ds(HW + g * L, L)] = hi * bv
            return carry2

        lax.fori_loop(0, CH, edge_body, 0, unroll=8)

    stage_start(0, 0)

    def sup_body(k, carry):
        b = lax.rem(k, 2)
        stage_wait(k, b)

        @pl.when(k < NSUP - 1)
        def _():
            stage_start(k + 1, 1 - b)

        compute_ex(k, b, 0, 0)
        gather_start(b, 0, 0)

        def pair_body(mm, carry1):
            for p in (0, 1):
                j = 2 * mm + p
                q = 1 - p
                gather_wait(b, j, p)
                scale(p)
                scatter_start(b, j, p)
                if p == 0:
                    @pl.when(mm > 0)
                    def _():
                        scatter_wait(b, j - 1, q)
                else:
                    scatter_wait(b, j - 1, q)

                if p == 0:
                    compute_ex(k, b, j + 1, q)
                    gather_start(b, j + 1, q)
                else:
                    @pl.when(mm < SUP // 2 - 1)
                    def _():
                        compute_ex(k, b, j + 1, q)
                        gather_start(b, j + 1, q)
            return carry1

        lax.fori_loop(0, SUP // 2, pair_body, 0)
        scatter_wait(b, SUP - 1, 1)
        return carry

    lax.fori_loop(0, NSUP, sup_body, 0)

    plsc.subcore_barrier()

    pltpu.sync_copy(att_sh.at[pl.ds(s * RPT, RPT)],
                    att_out.at[c, pl.ds(s * RPT, RPT)])
    pltpu.sync_copy(den_sh.at[pl.ds(s * RPT, RPT)],
                    den_out.at[c, pl.ds(s * RPT, RPT)])


def _fin_body(h_ref, a0_ref, a1_ref, d0_ref, d1_ref, w1t_ref, b1_ref,
              w2t_ref, b2_ref, g1_ref, be1_ref, g2_ref, be2_ref, out_ref):
    att = a0_ref[...] + a1_ref[...]
    den = jnp.maximum(d0_ref[...] + d1_ref[...], 1e-16)
    v = h_ref[...] + att / den
    mu = jnp.mean(v, axis=-1, keepdims=True)
    var = jnp.var(v, axis=-1, keepdims=True)
    h1 = (v - mu) / jnp.sqrt(var + 1e-5) * g1_ref[...] + be1_ref[...]
    ff = jnp.maximum(
        jnp.dot(h1, w1t_ref[...], preferred_element_type=jnp.float32)
        + b1_ref[...], 0.0)
    ff = jnp.dot(ff, w2t_ref[...],
                 preferred_element_type=jnp.float32) + b2_ref[...]
    v2 = h1 + ff
    mu2 = jnp.mean(v2, axis=-1, keepdims=True)
    var2 = jnp.var(v2, axis=-1, keepdims=True)
    out_ref[...] = ((v2 - mu2) / jnp.sqrt(var2 + 1e-5) * g2_ref[...]
                    + be2_ref[...])


def kernel(x, edge_index, edge_attr, W_node, att_w, ln1_g, ln1_b, W1, b1,
           W2, b2, ln2_g, ln2_b):
    f32 = jnp.float32

    a_dst, a_src, a_edge = att_w[:D], att_w[D:2 * D], att_w[2 * D:]
    a2 = jnp.zeros((D, 8), f32).at[:, 0].set(a_dst).at[:, 1].set(a_src)
    bd = jnp.zeros((16, 8), f32).at[:, 0].set(a_edge)

    B1 = 1000
    h, sds = pl.pallas_call(
        _proj_body,
        grid=(N // B1,),
        in_specs=[pl.BlockSpec((B1, D), lambda i: (i, 0)),
                  pl.BlockSpec((D, D), lambda i: (0, 0)),
                  pl.BlockSpec((D, 8), lambda i: (0, 0))],
        out_specs=[pl.BlockSpec((B1, D), lambda i: (i, 0)),
                   pl.BlockSpec((B1, 8), lambda i: (i, 0))],
        out_shape=[jax.ShapeDtypeStruct((N, D), f32),
                   jax.ShapeDtypeStruct((N, 8), f32)],
    )(x, W_node.T, a2)
    sd = sds[:, 0]
    ss = sds[:, 1]

    hb = jnp.stack([h[:, :HW], h[:, HW:]], axis=-1).astype(jnp.bfloat16)
    h_pack = lax.bitcast_convert_type(hb, jnp.int32)

    B2 = 4000
    ed_full = pl.pallas_call(
        _ed_body,
        grid=(E // B2,),
        in_specs=[pl.BlockSpec((B2, 16), lambda i: (i, 0)),
                  pl.BlockSpec((16, 8), lambda i: (0, 0))],
        out_specs=pl.BlockSpec((B2, 8), lambda i: (i, 0)),
        out_shape=jax.ShapeDtypeStruct((E, 8), f32),
    )(edge_attr, bd)
    ed_r = ed_full[:, 0]

    pad = E_PAD - E
    dst = jnp.pad(edge_index[0].astype(jnp.int32), (0, pad)).reshape(
        NW, NSUP, SUP, CH)
    src = jnp.pad(edge_index[1].astype(jnp.int32), (0, pad)).reshape(
        NW, NSUP, SUP, CH)
    ed = jnp.pad(ed_r, (0, pad)).reshape(NW, NSUP, SUP, CH)
    zatt = jnp.zeros((N_PAD, D), f32)
    zden = jnp.zeros((N_PAD,), f32)

    mesh = plsc.VectorSubcoreMesh(core_axis_name="c", subcore_axis_name="s",
                                  num_cores=NC, num_subcores=NS)
    att_acc, den_acc = pl.kernel(
        _sc_body,
        out_type=[jax.ShapeDtypeStruct((NC, N_PAD, D), f32),
                  jax.ShapeDtypeStruct((NC, N_PAD), f32)],
        mesh=mesh,
        compiler_params=pltpu.CompilerParams(needs_layout_passes=False,
                                             use_tc_tiling_on_sc=False),
        scratch_types=[
            pltpu.VMEM((N,), f32),
            pltpu.VMEM((N,), f32),
            pltpu.VMEM((2, SUP, CH), jnp.int32),
            pltpu.VMEM((2, SUP, CH), jnp.int32),
            pltpu.VMEM((2, SUP, CH), f32),
            pltpu.VMEM((2, CH), f32),
            pltpu.VMEM((2, CH, HW), jnp.int32),
            pltpu.VMEM((2, CH, D), f32),
            pltpu.VMEM_SHARED((N_PAD, D), f32),
            pltpu.VMEM_SHARED((N_PAD,), f32),
            pltpu.SemaphoreType.DMA,
            pltpu.SemaphoreType.DMA,
            pltpu.SemaphoreType.DMA,
            pltpu.SemaphoreType.DMA,
            pltpu.SemaphoreType.DMA,
        ],
    )(h_pack, sd, ss, dst, src, ed, zatt, zden)

    B3 = 1000
    out = pl.pallas_call(
        _fin_body,
        grid=(N // B3,),
        in_specs=[pl.BlockSpec((B3, D), lambda i: (i, 0)),
                  pl.BlockSpec((B3, D), lambda i: (i, 0)),
                  pl.BlockSpec((B3, D), lambda i: (i, 0)),
                  pl.BlockSpec((B3, 1), lambda i: (i, 0)),
                  pl.BlockSpec((B3, 1), lambda i: (i, 0)),
                  pl.BlockSpec((D, 2 * D), lambda i: (0, 0)),
                  pl.BlockSpec((1, 2 * D), lambda i: (0, 0)),
                  pl.BlockSpec((2 * D, D), lambda i: (0, 0)),
                  pl.BlockSpec((1, D), lambda i: (0, 0)),
                  pl.BlockSpec((1, D), lambda i: (0, 0)),
                  pl.BlockSpec((1, D), lambda i: (0, 0)),
                  pl.BlockSpec((1, D), lambda i: (0, 0)),
                  pl.BlockSpec((1, D), lambda i: (0, 0))],
        out_specs=pl.BlockSpec((B3, D), lambda i: (i, 0)),
        out_shape=jax.ShapeDtypeStruct((N, D), f32),
    )(h, att_acc[0, :N], att_acc[1, :N],
      den_acc[0, :N].reshape(N, 1), den_acc[1, :N].reshape(N, 1),
      W1.T, b1.reshape(1, -1), W2.T, b2.reshape(1, -1),
      ln1_g.reshape(1, -1), ln1_b.reshape(1, -1),
      ln2_g.reshape(1, -1), ln2_b.reshape(1, -1))
    return out


# speedup vs baseline: 1.2423x; 1.2423x over previous
import functools

import jax
import jax.numpy as jnp
from jax import lax
from jax.experimental import pallas as pl
from jax.experimental.pallas import tpu as pltpu
from jax.experimental.pallas import tpu_sc as plsc

N = 10000
E = 320000
D = 128
NC, NS, L = 2, 16, 16
NW = NC * NS
EPT = 10240
E_PAD = NW * EPT
CH = 64
SUP = 8
NSUP = EPT // (SUP * CH)
N_PAD = 10240
RPT = N_PAD // NS
HW = D // 2


def _proj_body(x_ref, wt_ref, a2_ref, h_ref, sds_ref):
    h = jnp.dot(x_ref[...], wt_ref[...], preferred_element_type=jnp.float32)
    h_ref[...] = h
    sds_ref[...] = jnp.dot(h, a2_ref[...], preferred_element_type=jnp.float32)


def _ed_body(ea_ref, bd_ref, ed_ref):
    ed_ref[...] = jnp.dot(ea_ref[...], bd_ref[...],
                          preferred_element_type=jnp.float32)


def _sc_body(h_hbm, sd_hbm, ss_hbm, dst_hbm, src_hbm, ed_hbm,
             zatt_hbm, zden_hbm,
             att_out, den_out,
             sd_v, ss_v, dst_v, src_v, ed_v, ex_v, rows_v, scaled_v,
             att_sh, den_sh,
             gsem0, gsem1, gsem2, gsem3, ssem0, ssem1, stsem):
    c = lax.axis_index("c")
    s = lax.axis_index("s")
    wid = s * NC + c

    pltpu.sync_copy(sd_hbm, sd_v)
    pltpu.sync_copy(ss_hbm, ss_v)
    pltpu.sync_copy(zatt_hbm.at[pl.ds(s * RPT, RPT)],
                    att_sh.at[pl.ds(s * RPT, RPT)])
    pltpu.sync_copy(zden_hbm.at[pl.ds(s * RPT, RPT)],
                    den_sh.at[pl.ds(s * RPT, RPT)])
    plsc.subcore_barrier()

    lane = lax.iota(jnp.int32, L)

    def stage_start(k, b):
        pltpu.async_copy(dst_hbm.at[wid, k], dst_v.at[b], stsem)
        pltpu.async_copy(src_hbm.at[wid, k], src_v.at[b], stsem)
        pltpu.async_copy(ed_hbm.at[wid, k], ed_v.at[b], stsem)

    def stage_wait(k, b):
        pltpu.make_async_copy(dst_hbm.at[wid, k], dst_v.at[b], stsem).wait()
        pltpu.make_async_copy(src_hbm.at[wid, k], src_v.at[b], stsem).wait()
        pltpu.make_async_copy(ed_hbm.at[wid, k], ed_v.at[b], stsem).wait()

    def gsem(p):
        return gsem0 if p == 0 else gsem1

    def gsemh(p):
        return gsem2 if p == 0 else gsem3

    def ssem(p):
        return ssem0 if p == 0 else ssem1

    def compute_ex(k, b, j, p):
        for g in range(CH // L):
            di = dst_v[b, j, pl.ds(g * L, L)]
            si = src_v[b, j, pl.ds(g * L, L)]
            sc = (plsc.load_gather(sd_v, [di])
                  + plsc.load_gather(ss_v, [si])
                  + ed_v[b, j, pl.ds(g * L, L)])
            sc = jnp.where(sc >= 0.0, sc, 0.2 * sc)
            ex = jnp.exp(sc)
            gid = wid * EPT + (k * SUP + j) * CH + g * L + lane
            ex_v[p, pl.ds(g * L, L)] = jnp.where(gid < E, ex, 0.0)

    HC = CH // 2

    def gather_start(b, j, p):
        pltpu.async_copy(h_hbm.at[src_v.at[b, j, pl.ds(0, HC)]],
                         rows_v.at[p, pl.ds(0, HC)], gsem(p))
        pltpu.async_copy(h_hbm.at[src_v.at[b, j, pl.ds(HC, HC)]],
                         rows_v.at[p, pl.ds(HC, HC)], gsemh(p))

    def gather_wait(b, j, p):
        pltpu.make_async_copy(
            h_hbm.at[src_v.at[b, j, pl.ds(0, HC)]],
            rows_v.at[p, pl.ds(0, HC)], gsem(p)).wait()
        pltpu.make_async_copy(
            h_hbm.at[src_v.at[b, j, pl.ds(HC, HC)]],
            rows_v.at[p, pl.ds(HC, HC)], gsemh(p)).wait()

    def scatter_start(b, j, p):
        pltpu.async_copy(scaled_v.at[p], att_sh.at[dst_v.at[b, j]], ssem(p),
                         add=True)
        pltpu.async_copy(ex_v.at[p], den_sh.at[dst_v.at[b, j]], ssem(p),
                         add=True)

    def scatter_wait(b, j, p):
        pltpu.make_async_copy(
            scaled_v.at[p], att_sh.at[dst_v.at[b, j]], ssem(p)).wait()
        pltpu.make_async_copy(
            ex_v.at[p], den_sh.at[dst_v.at[b, j]], ssem(p)).wait()

    def scale(p):
        def edge_body(e, carry2):
            bv = plsc.load_gather(ex_v.at[p], [jnp.zeros((L,), jnp.int32) + e])
            for g in range(HW // L):
                v = rows_v[p, e, pl.ds(g * L, L)]
                lo = lax.bitcast_convert_type(v << 16, jnp.float32)
                hi = lax.bitcast_convert_type(v & jnp.int32(-65536),
                                              jnp.float32)
                scaled_v[p, e, pl.ds(g * L, L)] = lo * bv
                scaled_v[p, e, pl.ds(HW + g * L, L)] = hi * bv
            return carry2

        lax.fori_loop(0, CH, edge_body, 0, unroll=8)

    stage_start(0, 0)

    def sup_body(k, carry):
        b = lax.rem(k, 2)
        stage_wait(k, b)

        @pl.when(k < NSUP - 1)
        def _():
            stage_start(k + 1, 1 - b)

        compute_ex(k, b, 0, 0)
        gather_start(b, 0, 0)

        def pair_body(mm, carry1):
            for p in (0, 1):
                j = 2 * mm + p
                q = 1 - p
                gather_wait(b, j, p)
                scale(p)
                scatter_start(b, j, p)
                if p == 0:
                    @pl.when(mm > 0)
                    def _():
                        scatter_wait(b, j - 1, q)
                else:
                    scatter_wait(b, j - 1, q)

                if p == 0:
                    compute_ex(k, b, j + 1, q)
                    gather_start(b, j + 1, q)
                else:
                    @pl.when(mm < SUP // 2 - 1)
                    def _():
                        compute_ex(k, b, j + 1, q)
                        gather_start(b, j + 1, q)
            return carry1

        lax.fori_loop(0, SUP // 2, pair_body, 0)
        scatter_wait(b, SUP - 1, 1)
        return carry

    lax.fori_loop(0, NSUP, sup_body, 0)

    plsc.subcore_barrier()

    pltpu.sync_copy(att_sh.at[pl.ds(s * RPT, RPT)],
                    att_out.at[c, pl.ds(s * RPT, RPT)])
    pltpu.sync_copy(den_sh.at[pl.ds(s * RPT, RPT)],
                    den_out.at[c, pl.ds(s * RPT, RPT)])


def _fin_body(h_ref, a0_ref, a1_ref, d0_ref, d1_ref, w1t_ref, b1_ref,
              w2t_ref, b2_ref, g1_ref, be1_ref, g2_ref, be2_ref, out_ref):
    att = a0_ref[...] + a1_ref[...]
    den = jnp.maximum(d0_ref[...] + d1_ref[...], 1e-16)
    v = h_ref[...] + att / den
    mu = jnp.mean(v, axis=-1, keepdims=True)
    var = jnp.var(v, axis=-1, keepdims=True)
    h1 = (v - mu) / jnp.sqrt(var + 1e-5) * g1_ref[...] + be1_ref[...]
    ff = jnp.maximum(
        jnp.dot(h1, w1t_ref[...], preferred_element_type=jnp.float32)
        + b1_ref[...], 0.0)
    ff = jnp.dot(ff, w2t_ref[...],
                 preferred_element_type=jnp.float32) + b2_ref[...]
    v2 = h1 + ff
    mu2 = jnp.mean(v2, axis=-1, keepdims=True)
    var2 = jnp.var(v2, axis=-1, keepdims=True)
    out_ref[...] = ((v2 - mu2) / jnp.sqrt(var2 + 1e-5) * g2_ref[...]
                    + be2_ref[...])


def kernel(x, edge_index, edge_attr, W_node, att_w, ln1_g, ln1_b, W1, b1,
           W2, b2, ln2_g, ln2_b):
    f32 = jnp.float32

    a_dst, a_src, a_edge = att_w[:D], att_w[D:2 * D], att_w[2 * D:]
    a2 = jnp.zeros((D, 8), f32).at[:, 0].set(a_dst).at[:, 1].set(a_src)
    bd = jnp.zeros((D, 8), f32)
    for g in range(8):
        bd = bd.at[16 * g:16 * (g + 1), g].set(a_edge)

    B1 = 1000
    h, sds = pl.pallas_call(
        _proj_body,
        grid=(N // B1,),
        in_specs=[pl.BlockSpec((B1, D), lambda i: (i, 0)),
                  pl.BlockSpec((D, D), lambda i: (0, 0)),
                  pl.BlockSpec((D, 8), lambda i: (0, 0))],
        out_specs=[pl.BlockSpec((B1, D), lambda i: (i, 0)),
                   pl.BlockSpec((B1, 8), lambda i: (i, 0))],
        out_shape=[jax.ShapeDtypeStruct((N, D), f32),
                   jax.ShapeDtypeStruct((N, 8), f32)],
    )(x, W_node.T, a2)
    sd = sds[:, 0]
    ss = sds[:, 1]

    hb = jnp.stack([h[:, :HW], h[:, HW:]], axis=-1).astype(jnp.bfloat16)
    h_pack = lax.bitcast_convert_type(hb, jnp.int32)

    B2 = 1000
    ea_r = edge_attr.reshape(E // 8, D)
    ed_r = pl.pallas_call(
        _ed_body,
        grid=(E // 8 // B2,),
        in_specs=[pl.BlockSpec((B2, D), lambda i: (i, 0)),
                  pl.BlockSpec((D, 8), lambda i: (0, 0))],
        out_specs=pl.BlockSpec((B2, 8), lambda i: (i, 0)),
        out_shape=jax.ShapeDtypeStruct((E // 8, 8), f32),
    )(ea_r, bd).reshape(E)

    pad = E_PAD - E
    dst = jnp.pad(edge_index[0].astype(jnp.int32), (0, pad)).reshape(
        NW, NSUP, SUP, CH)
    src = jnp.pad(edge_index[1].astype(jnp.int32), (0, pad)).reshape(
        NW, NSUP, SUP, CH)
    ed = jnp.pad(ed_r, (0, pad)).reshape(NW, NSUP, SUP, CH)
    zatt = jnp.zeros((N_PAD, D), f32)
    zden = jnp.zeros((N_PAD,), f32)

    mesh = plsc.VectorSubcoreMesh(core_axis_name="c", subcore_axis_name="s",
                                  num_cores=NC, num_subcores=NS)
    att_acc, den_acc = pl.kernel(
        _sc_body,
        out_type=[jax.ShapeDtypeStruct((NC, N_PAD, D), f32),
                  jax.ShapeDtypeStruct((NC, N_PAD), f32)],
        mesh=mesh,
        compiler_params=pltpu.CompilerParams(needs_layout_passes=False,
                                             use_tc_tiling_on_sc=False),
        scratch_types=[
            pltpu.VMEM((N,), f32),
            pltpu.VMEM((N,), f32),
            pltpu.VMEM((2, SUP, CH), jnp.int32),
            pltpu.VMEM((2, SUP, CH), jnp.int32),
            pltpu.VMEM((2, SUP, CH), f32),
            pltpu.VMEM((2, CH), f32),
            pltpu.VMEM((2, CH, HW), jnp.int32),
            pltpu.VMEM((2, CH, D), f32),
            pltpu.VMEM_SHARED((N_PAD, D), f32),
            pltpu.VMEM_SHARED((N_PAD,), f32),
            pltpu.SemaphoreType.DMA,
            pltpu.SemaphoreType.DMA,
            pltpu.SemaphoreType.DMA,
            pltpu.SemaphoreType.DMA,
            pltpu.SemaphoreType.DMA,
            pltpu.SemaphoreType.DMA,
            pltpu.SemaphoreType.DMA,
        ],
    )(h_pack, sd, ss, dst, src, ed, zatt, zden)

    B3 = 1000
    out = pl.pallas_call(
        _fin_body,
        grid=(N // B3,),
        in_specs=[pl.BlockSpec((B3, D), lambda i: (i, 0)),
                  pl.BlockSpec((B3, D), lambda i: (i, 0)),
                  pl.BlockSpec((B3, D), lambda i: (i, 0)),
                  pl.BlockSpec((B3, 1), lambda i: (i, 0)),
                  pl.BlockSpec((B3, 1), lambda i: (i, 0)),
                  pl.BlockSpec((D, 2 * D), lambda i: (0, 0)),
                  pl.BlockSpec((1, 2 * D), lambda i: (0, 0)),
                  pl.BlockSpec((2 * D, D), lambda i: (0, 0)),
                  pl.BlockSpec((1, D), lambda i: (0, 0)),
                  pl.BlockSpec((1, D), lambda i: (0, 0)),
                  pl.BlockSpec((1, D), lambda i: (0, 0)),
                  pl.BlockSpec((1, D), lambda i: (0, 0)),
                  pl.BlockSpec((1, D), lambda i: (0, 0))],
        out_specs=pl.BlockSpec((B3, D), lambda i: (i, 0)),
        out_shape=jax.ShapeDtypeStruct((N, D), f32),
    )(h, att_acc[0, :N], att_acc[1, :N],
      den_acc[0, :N].reshape(N, 1), den_acc[1, :N].reshape(N, 1),
      W1.T, b1.reshape(1, -1), W2.T, b2.reshape(1, -1),
      ln1_g.reshape(1, -1), ln1_b.reshape(1, -1),
      ln2_g.reshape(1, -1), ln2_b.reshape(1, -1))
    return out
